# baseline (speedup 1.0000x reference)
import functools

import jax
import jax.numpy as jnp
from jax import lax
from jax.experimental import pallas as pl
from jax.experimental.pallas import tpu as pltpu
from jax.experimental.pallas import tpu_sc as plsc

N = 10000
NP = 10240
E = 160000
D = 300
NQ = 4
QD = 75
PQ = 80
DE = 16
L = 5
G = 256
BN = 2560

NTILE = 16
NCORE = 2
CH = 84
CHP = 44
NBUF = 4
NROWS = NP // NTILE
PRE_C = 24

PREC = jax.lax.Precision.DEFAULT

_SC_MESH = dict(core_axis_name="c", subcore_axis_name="s",
                num_cores=NCORE, num_subcores=NTILE)


def _scatter_pipeline(gat, acc, dst_v, bufs, gsems, ssems, n_chunks):
    for b in range(NBUF):
        pltpu.async_copy(gat(b), bufs[b], gsems[b])

    def step(jj, carry):
        j = jj * NBUF
        for b in range(NBUF):
            pltpu.make_async_copy(gat(0), bufs[b], gsems[b]).wait()
            pltpu.async_copy(bufs[b], acc.at[dst_v.at[j + b]], ssems[b],
                             add=True)
        for b in range(NBUF):
            pltpu.make_async_copy(bufs[b], acc.at[pl.ds(0, 128)],
                                  ssems[b]).wait()
            pltpu.async_copy(gat(j + NBUF + b), bufs[b], gsems[b])
        return carry

    lax.fori_loop(0, (n_chunks - NBUF) // NBUF, step, 0)
    j0 = n_chunks - NBUF
    for b in range(NBUF):
        pltpu.make_async_copy(gat(0), bufs[b], gsems[b]).wait()
        pltpu.async_copy(bufs[b], acc.at[dst_v.at[j0 + b]], ssems[b],
                         add=True)
    for b in range(NBUF):
        pltpu.make_async_copy(bufs[b], acc.at[pl.ds(0, 128)],
                              ssems[b]).wait()


def _spmv_body(h_hbm, src_hbm, dst_hbm, out_hbm,
               acc, src_v, dst_v, b0, b1, b2, b3,
               g0, g1, g2, g3, s0, s1, s2, s3):
    c = lax.axis_index("c")
    s = lax.axis_index("s")
    pltpu.sync_copy(src_hbm.at[s], src_v)
    pltpu.sync_copy(dst_hbm.at[s], dst_v)

    for q in range(2):
        qq = 2 * c + q
        hq = h_hbm.at[qq]
        pltpu.sync_copy(hq.at[pl.ds(s * NROWS, NROWS)],
                        acc.at[pl.ds(s * NROWS, NROWS)])
        plsc.subcore_barrier()

        _scatter_pipeline(lambda j: hq.at[src_v.at[j]], acc, dst_v,
                          (b0, b1, b2, b3), (g0, g1, g2, g3),
                          (s0, s1, s2, s3), CH)

        plsc.subcore_barrier()
        pltpu.sync_copy(acc.at[pl.ds(s * NROWS, NROWS)],
                        out_hbm.at[qq, pl.ds(s * NROWS, NROWS)])


_spmv_call = pl.kernel(
    _spmv_body,
    out_type=jax.ShapeDtypeStruct((NQ, NP, PQ), jnp.float32),
    compiler_params=pltpu.CompilerParams(use_tc_tiling_on_sc=False),
    mesh=plsc.VectorSubcoreMesh(**_SC_MESH),
    scratch_types=[
        pltpu.VMEM_SHARED((NP, PQ), jnp.float32),
        pltpu.VMEM((CH, 128), jnp.int32),
        pltpu.VMEM((CH, 128), jnp.int32),
    ] + [pltpu.VMEM((128, PQ), jnp.float32)] * NBUF
      + [pltpu.SemaphoreType.DMA] * (2 * NBUF),
)


def _pre_body(ea_hbm, dst_hbm, zero_hbm, out_hbm,
              acc, dst_v, b0, b1, b2, b3,
              g0, g1, g2, g3, s0, s1, s2, s3):
    c = lax.axis_index("c")
    s = lax.axis_index("s")
    t = c * NTILE + s
    base = t * (CHP * 128)
    pltpu.sync_copy(dst_hbm.at[t], dst_v)
    pltpu.sync_copy(zero_hbm.at[pl.ds(s * NROWS, NROWS)],
                    acc.at[pl.ds(s * NROWS, NROWS)])
    plsc.subcore_barrier()

    _scatter_pipeline(lambda j: ea_hbm.at[pl.ds(base + j * 128, 128)],
                      acc, dst_v, (b0, b1, b2, b3), (g0, g1, g2, g3),
                      (s0, s1, s2, s3), CHP)

    plsc.subcore_barrier()
    pltpu.sync_copy(acc.at[pl.ds(s * NROWS, NROWS)],
                    out_hbm.at[c, pl.ds(s * NROWS, NROWS)])


_pre_call = pl.kernel(
    _pre_body,
    out_type=jax.ShapeDtypeStruct((NCORE, NP, PRE_C), jnp.float32),
    compiler_params=pltpu.CompilerParams(use_tc_tiling_on_sc=False),
    mesh=plsc.VectorSubcoreMesh(**_SC_MESH),
    scratch_types=[
        pltpu.VMEM_SHARED((NP, PRE_C), jnp.float32),
        pltpu.VMEM((CHP, 128), jnp.int32),
    ] + [pltpu.VMEM((128, PRE_C), jnp.float32)] * NBUF
      + [pltpu.SemaphoreType.DMA] * (2 * NBUF),
)


def _dot(a, b):
    return jax.lax.dot_general(a, b, (((1,), (0,)), ((), ())),
                               preferred_element_type=jnp.float32,
                               precision=PREC)


def _layer_body(s_ref, ae_ref, we_ref, be_ref, w1_ref, b1_ref, w2_ref, b2_ref,
                out_ref, *, relu_out):
    u = jnp.concatenate([s_ref[q, :, :QD] for q in range(NQ)], axis=1)
    aedeg = ae_ref[0] + ae_ref[1]
    ae = aedeg[:, :DE]
    deg = aedeg[:, DE:DE + 1]
    u = u + _dot(ae, we_ref[...]) + deg * be_ref[...]
    t = jnp.maximum(_dot(u, w1_ref[...]) + b1_ref[...], 0.0)
    v = _dot(t, w2_ref[...]) + b2_ref[...]
    if relu_out:
        v = jnp.maximum(v, 0.0)
    z = jnp.zeros((BN, PQ - QD), jnp.float32)
    out_ref[...] = jnp.stack(
        [jnp.concatenate([v[:, q * QD:(q + 1) * QD], z], axis=1)
         for q in range(NQ)])


def _layer_call(s_split, aedeg, we, be, w1, b1, w2, b2, relu_out):
    return pl.pallas_call(
        functools.partial(_layer_body, relu_out=relu_out),
        grid=(NP // BN,),
        in_specs=[
            pl.BlockSpec((NQ, BN, PQ), lambda i: (0, i, 0)),
            pl.BlockSpec((2, BN, PRE_C), lambda i: (0, i, 0)),
            pl.BlockSpec((DE, D), lambda i: (0, 0)),
            pl.BlockSpec((1, D), lambda i: (0, 0)),
            pl.BlockSpec((D, 2 * D), lambda i: (0, 0)),
            pl.BlockSpec((1, 2 * D), lambda i: (0, 0)),
            pl.BlockSpec((2 * D, D), lambda i: (0, 0)),
            pl.BlockSpec((1, D), lambda i: (0, 0)),
        ],
        out_specs=pl.BlockSpec((NQ, BN, PQ), lambda i: (0, i, 0)),
        out_shape=jax.ShapeDtypeStruct((NQ, NP, PQ), jnp.float32),
    )(s_split, aedeg, we, be.reshape(1, D), w1, b1.reshape(1, 2 * D),
      w2, b2.reshape(1, D))


def _pool_body(h_ref, b_ref, wc1_ref, bc1_ref, wc2_ref, bc2_ref,
               out_ref, acc_ref):
    i = pl.program_id(0)

    @pl.when(i == 0)
    def _():
        acc_ref[...] = jnp.zeros_like(acc_ref)

    h = jnp.concatenate([h_ref[q, :, :QD] for q in range(NQ)]
                        + [jnp.ones((BN, 1), jnp.float32),
                           jnp.zeros((BN, 3), jnp.float32)], axis=1)
    oh = (b_ref[...] == jax.lax.broadcasted_iota(jnp.int32, (BN, G), 1)
          ).astype(jnp.float32)
    acc_ref[...] += jax.lax.dot_general(
        oh, h, (((0,), (0,)), ((), ())),
        preferred_element_type=jnp.float32, precision=PREC)

    @pl.when(i == NP // BN - 1)
    def _():
        acc = acc_ref[...]
        pooled = acc[:, :D] / jnp.maximum(acc[:, D:D + 1], 1.0)
        z = jnp.maximum(_dot(pooled, wc1_ref[...]) + bc1_ref[...], 0.0)
        out_ref[...] = _dot(z, wc2_ref[...]) + bc2_ref[...]


def _pool_call(h_split, batch2d, wc1, bc1, wc2, bc2):
    return pl.pallas_call(
        _pool_body,
        grid=(NP // BN,),
        in_specs=[
            pl.BlockSpec((NQ, BN, PQ), lambda i: (0, i, 0)),
            pl.BlockSpec((BN, 1), lambda i: (i, 0)),
            pl.BlockSpec((D, D), lambda i: (0, 0)),
            pl.BlockSpec((1, D), lambda i: (0, 0)),
            pl.BlockSpec((D, 1), lambda i: (0, 0)),
            pl.BlockSpec((1, 1), lambda i: (0, 0)),
        ],
        out_specs=pl.BlockSpec((G, 1), lambda i: (0, 0)),
        out_shape=jax.ShapeDtypeStruct((G, 1), jnp.float32),
        scratch_shapes=[pltpu.VMEM((G, D + 4), jnp.float32)],
    )(h_split, batch2d, wc1, bc1.reshape(1, D), wc2, bc2.reshape(1, 1))


def _to_split(h):
    hp = jnp.pad(h, ((0, NP - N), (0, 0)))
    return jnp.stack([jnp.pad(hp[:, q * QD:(q + 1) * QD],
                              ((0, 0), (0, PQ - QD))) for q in range(NQ)])


def _encode_prep(x, ea, ei):
    src, dst = ei[0], ei[1]

    npad = NTILE * CH * 128 - E
    pad_i = jnp.arange(npad, dtype=jnp.int32)
    srcp = jnp.concatenate([src, N + pad_i % (NP - N)])
    dstp = jnp.concatenate([dst, pad_i % N])
    src3 = srcp.reshape(NTILE, CH, 128)
    dst3 = dstp.reshape(NTILE, CH, 128)

    npad2 = NCORE * NTILE * CHP * 128 - E
    pad_i2 = jnp.arange(npad2, dtype=jnp.int32)
    dstp2 = jnp.concatenate([dst, pad_i2 % N])
    dst3p = dstp2.reshape(NCORE * NTILE, CHP, 128)
    ea2 = jnp.concatenate(
        [ea, jnp.ones((E, 1), jnp.float32), jnp.zeros((E, PRE_C - DE - 1),
                                                      jnp.float32)], axis=1)
    ea2p = jnp.concatenate(
        [ea2, jnp.zeros((npad2, PRE_C), jnp.float32)], axis=0)
    zeros32 = jnp.zeros((NP, PRE_C), jnp.float32)

    aedeg = _pre_call(ea2p, dst3p, zeros32)
    return _to_split(x), src3, dst3, aedeg


def _finish(h_split, batch, Wc1, bc1, Wc2, bc2):
    batch2d = jnp.pad(batch[:, None], ((0, NP - N), (0, 0)),
                      constant_values=G)
    logits = _pool_call(h_split, batch2d, Wc1, bc1, Wc2, bc2)
    return logits[:, 0]


@jax.jit
def kernel(x0, edge_attr0, x1, edge_attr1, W1, b1, W2, b2, We, be,
           Wc1, bc1, Wc2, bc2, edge_index0, batch0, edge_index1, batch1):
    ha, src3a, dst3a, aedega = _encode_prep(x0, edge_attr0, edge_index0)
    hb, src3b, dst3b, aedegb = _encode_prep(x1, edge_attr1, edge_index1)
    for l in range(L):
        sa = _spmv_call(ha, src3a, dst3a)
        sb = _spmv_call(hb, src3b, dst3b)
        ha = _layer_call(sa, aedega, We[l], be[l], W1[l], b1[l],
                         W2[l], b2[l], relu_out=(l < L - 1))
        hb = _layer_call(sb, aedegb, We[l], be[l], W1[l], b1[l],
                         W2[l], b2[l], relu_out=(l < L - 1))
    pos = _finish(ha, batch0, Wc1, bc1, Wc2, bc2)
    neg = _finish(hb, batch1, Wc1, bc1, Wc2, bc2)
    logits = jnp.concatenate([pos, neg], axis=0)
    labels = jnp.concatenate([jnp.ones((G,), jnp.float32),
                              jnp.zeros((G,), jnp.float32)], axis=0)
    return logits, labels


# speedup vs baseline: 1.0382x; 1.0382x over previous
import functools

import jax
import jax.numpy as jnp
from jax import lax
from jax.experimental import pallas as pl
from jax.experimental.pallas import tpu as pltpu
from jax.experimental.pallas import tpu_sc as plsc

N = 10000
NP = 10240
E = 160000
D = 300
NQ = 4
QD = 75
PQ = 80
DE = 16
L = 5
G = 256
BN = 2560

NTILE = 16
NCORE = 2
CH = 80
CHP = 40
NBUF = 4
NROWS = NP // NTILE
PRE_C = 24

PREC = jax.lax.Precision.DEFAULT

_SC_MESH = dict(core_axis_name="c", subcore_axis_name="s",
                num_cores=NCORE, num_subcores=NTILE)


def _scatter_pipeline(gat, acc, dst_v, bufs, gsems, ssems, n_chunks):
    for b in range(NBUF):
        pltpu.async_copy(gat(b), bufs[b], gsems[b])

    def step(jj, carry):
        j = jj * NBUF
        for b in range(NBUF):
            pltpu.make_async_copy(gat(0), bufs[b], gsems[b]).wait()
            pltpu.async_copy(bufs[b], acc.at[dst_v.at[j + b]], ssems[b],
                             add=True)
        for b in range(NBUF):
            pltpu.make_async_copy(bufs[b], acc.at[pl.ds(0, 128)],
                                  ssems[b]).wait()
            pltpu.async_copy(gat(j + NBUF + b), bufs[b], gsems[b])
        return carry

    lax.fori_loop(0, (n_chunks - NBUF) // NBUF, step, 0)
    j0 = n_chunks - NBUF
    for b in range(NBUF):
        pltpu.make_async_copy(gat(0), bufs[b], gsems[b]).wait()
        pltpu.async_copy(bufs[b], acc.at[dst_v.at[j0 + b]], ssems[b],
                         add=True)
    for b in range(NBUF):
        pltpu.make_async_copy(bufs[b], acc.at[pl.ds(0, 128)],
                              ssems[b]).wait()


def _spmv_body(h_hbm, src_hbm, dst_hbm, out_hbm,
               acc, src_v, dst_v, b0, b1, b2, b3,
               g0, g1, g2, g3, s0, s1, s2, s3):
    c = lax.axis_index("c")
    s = lax.axis_index("s")
    pltpu.sync_copy(src_hbm.at[s], src_v)
    pltpu.sync_copy(dst_hbm.at[s], dst_v)

    for q in range(2):
        qq = 2 * c + q
        hq = h_hbm.at[qq]
        pltpu.sync_copy(hq.at[pl.ds(s * NROWS, NROWS)],
                        acc.at[pl.ds(s * NROWS, NROWS)])
        plsc.subcore_barrier()

        _scatter_pipeline(lambda j: hq.at[src_v.at[j]], acc, dst_v,
                          (b0, b1, b2, b3), (g0, g1, g2, g3),
                          (s0, s1, s2, s3), CH)

        plsc.subcore_barrier()
        pltpu.sync_copy(acc.at[pl.ds(s * NROWS, NROWS)],
                        out_hbm.at[qq, pl.ds(s * NROWS, NROWS)])


_spmv_call = pl.kernel(
    _spmv_body,
    out_type=jax.ShapeDtypeStruct((NQ, NP, PQ), jnp.float32),
    compiler_params=pltpu.CompilerParams(use_tc_tiling_on_sc=False),
    mesh=plsc.VectorSubcoreMesh(**_SC_MESH),
    scratch_types=[
        pltpu.VMEM_SHARED((NP, PQ), jnp.float32),
        pltpu.VMEM((CH, 128), jnp.int32),
        pltpu.VMEM((CH, 128), jnp.int32),
    ] + [pltpu.VMEM((128, PQ), jnp.float32)] * NBUF
      + [pltpu.SemaphoreType.DMA] * (2 * NBUF),
)


def _pre_body(ea_hbm, dst_hbm, zero_hbm, out_hbm,
              acc, dst_v, b0, b1, b2, b3,
              g0, g1, g2, g3, s0, s1, s2, s3):
    c = lax.axis_index("c")
    s = lax.axis_index("s")
    t = c * NTILE + s
    base = t * (CHP * 128)
    pltpu.sync_copy(dst_hbm.at[t], dst_v)
    pltpu.sync_copy(zero_hbm.at[pl.ds(s * NROWS, NROWS)],
                    acc.at[pl.ds(s * NROWS, NROWS)])
    plsc.subcore_barrier()

    _scatter_pipeline(lambda j: ea_hbm.at[pl.ds(base + j * 128, 128)],
                      acc, dst_v, (b0, b1, b2, b3), (g0, g1, g2, g3),
                      (s0, s1, s2, s3), CHP)

    plsc.subcore_barrier()
    pltpu.sync_copy(acc.at[pl.ds(s * NROWS, NROWS)],
                    out_hbm.at[c, pl.ds(s * NROWS, NROWS)])


_pre_call = pl.kernel(
    _pre_body,
    out_type=jax.ShapeDtypeStruct((NCORE, NP, PRE_C), jnp.float32),
    compiler_params=pltpu.CompilerParams(use_tc_tiling_on_sc=False),
    mesh=plsc.VectorSubcoreMesh(**_SC_MESH),
    scratch_types=[
        pltpu.VMEM_SHARED((NP, PRE_C), jnp.float32),
        pltpu.VMEM((CHP, 128), jnp.int32),
    ] + [pltpu.VMEM((128, PRE_C), jnp.float32)] * NBUF
      + [pltpu.SemaphoreType.DMA] * (2 * NBUF),
)


def _dot(a, b):
    return jax.lax.dot_general(a, b, (((1,), (0,)), ((), ())),
                               preferred_element_type=jnp.float32,
                               precision=PREC)


def _layer_body(s_ref, ae_ref, we_ref, be_ref, w1_ref, b1_ref, w2_ref, b2_ref,
                out_ref, *, relu_out):
    u = jnp.concatenate([s_ref[q, :, :QD] for q in range(NQ)], axis=1)
    aedeg = ae_ref[0] + ae_ref[1]
    ae = aedeg[:, :DE]
    deg = aedeg[:, DE:DE + 1]
    u = u + _dot(ae, we_ref[...]) + deg * be_ref[...]
    t = jnp.maximum(_dot(u, w1_ref[...]) + b1_ref[...], 0.0)
    v = _dot(t, w2_ref[...]) + b2_ref[...]
    if relu_out:
        v = jnp.maximum(v, 0.0)
    z = jnp.zeros((BN, PQ - QD), jnp.float32)
    out_ref[...] = jnp.stack(
        [jnp.concatenate([v[:, q * QD:(q + 1) * QD], z], axis=1)
         for q in range(NQ)])


def _layer_call(s_split, aedeg, we, be, w1, b1, w2, b2, relu_out):
    return pl.pallas_call(
        functools.partial(_layer_body, relu_out=relu_out),
        grid=(NP // BN,),
        in_specs=[
            pl.BlockSpec((NQ, BN, PQ), lambda i: (0, i, 0)),
            pl.BlockSpec((2, BN, PRE_C), lambda i: (0, i, 0)),
            pl.BlockSpec((DE, D), lambda i: (0, 0)),
            pl.BlockSpec((1, D), lambda i: (0, 0)),
            pl.BlockSpec((D, 2 * D), lambda i: (0, 0)),
            pl.BlockSpec((1, 2 * D), lambda i: (0, 0)),
            pl.BlockSpec((2 * D, D), lambda i: (0, 0)),
            pl.BlockSpec((1, D), lambda i: (0, 0)),
        ],
        out_specs=pl.BlockSpec((NQ, BN, PQ), lambda i: (0, i, 0)),
        out_shape=jax.ShapeDtypeStruct((NQ, NP, PQ), jnp.float32),
    )(s_split, aedeg, we, be.reshape(1, D), w1, b1.reshape(1, 2 * D),
      w2, b2.reshape(1, D))


def _pool_body(h_ref, b_ref, wc1_ref, bc1_ref, wc2_ref, bc2_ref,
               out_ref, acc_ref):
    i = pl.program_id(0)

    @pl.when(i == 0)
    def _():
        acc_ref[...] = jnp.zeros_like(acc_ref)

    h = jnp.concatenate([h_ref[q, :, :QD] for q in range(NQ)]
                        + [jnp.ones((BN, 1), jnp.float32),
                           jnp.zeros((BN, 3), jnp.float32)], axis=1)
    oh = (b_ref[...] == jax.lax.broadcasted_iota(jnp.int32, (BN, G), 1)
          ).astype(jnp.float32)
    acc_ref[...] += jax.lax.dot_general(
        oh, h, (((0,), (0,)), ((), ())),
        preferred_element_type=jnp.float32, precision=PREC)

    @pl.when(i == NP // BN - 1)
    def _():
        acc = acc_ref[...]
        pooled = acc[:, :D] / jnp.maximum(acc[:, D:D + 1], 1.0)
        z = jnp.maximum(_dot(pooled, wc1_ref[...]) + bc1_ref[...], 0.0)
        out_ref[...] = _dot(z, wc2_ref[...]) + bc2_ref[...]


def _pool_call(h_split, batch2d, wc1, bc1, wc2, bc2):
    return pl.pallas_call(
        _pool_body,
        grid=(NP // BN,),
        in_specs=[
            pl.BlockSpec((NQ, BN, PQ), lambda i: (0, i, 0)),
            pl.BlockSpec((BN, 1), lambda i: (i, 0)),
            pl.BlockSpec((D, D), lambda i: (0, 0)),
            pl.BlockSpec((1, D), lambda i: (0, 0)),
            pl.BlockSpec((D, 1), lambda i: (0, 0)),
            pl.BlockSpec((1, 1), lambda i: (0, 0)),
        ],
        out_specs=pl.BlockSpec((G, 1), lambda i: (0, 0)),
        out_shape=jax.ShapeDtypeStruct((G, 1), jnp.float32),
        scratch_shapes=[pltpu.VMEM((G, D + 4), jnp.float32)],
    )(h_split, batch2d, wc1, bc1.reshape(1, D), wc2, bc2.reshape(1, 1))


def _to_split(h):
    hp = jnp.pad(h, ((0, NP - N), (0, 0)))
    return jnp.stack([jnp.pad(hp[:, q * QD:(q + 1) * QD],
                              ((0, 0), (0, PQ - QD))) for q in range(NQ)])


def _encode_prep(x, ea, ei):
    src, dst = ei[0], ei[1]

    npad = NTILE * CH * 128 - E
    pad_i = jnp.arange(npad, dtype=jnp.int32)
    srcp = jnp.concatenate([src, N + pad_i % (NP - N)])
    dstp = jnp.concatenate([dst, pad_i % N])
    src3 = srcp.reshape(NTILE, CH, 128)
    dst3 = dstp.reshape(NTILE, CH, 128)

    npad2 = NCORE * NTILE * CHP * 128 - E
    pad_i2 = jnp.arange(npad2, dtype=jnp.int32)
    dstp2 = jnp.concatenate([dst, pad_i2 % N])
    dst3p = dstp2.reshape(NCORE * NTILE, CHP, 128)
    ea2 = jnp.concatenate(
        [ea, jnp.ones((E, 1), jnp.float32), jnp.zeros((E, PRE_C - DE - 1),
                                                      jnp.float32)], axis=1)
    ea2p = jnp.concatenate(
        [ea2, jnp.zeros((npad2, PRE_C), jnp.float32)], axis=0)
    zeros32 = jnp.zeros((NP, PRE_C), jnp.float32)

    aedeg = _pre_call(ea2p, dst3p, zeros32)
    return _to_split(x), src3, dst3, aedeg


def _finish(h_split, batch, Wc1, bc1, Wc2, bc2):
    batch2d = jnp.pad(batch[:, None], ((0, NP - N), (0, 0)),
                      constant_values=G)
    logits = _pool_call(h_split, batch2d, Wc1, bc1, Wc2, bc2)
    return logits[:, 0]


@jax.jit
def kernel(x0, edge_attr0, x1, edge_attr1, W1, b1, W2, b2, We, be,
           Wc1, bc1, Wc2, bc2, edge_index0, batch0, edge_index1, batch1):
    ha, src3a, dst3a, aedega = _encode_prep(x0, edge_attr0, edge_index0)
    hb, src3b, dst3b, aedegb = _encode_prep(x1, edge_attr1, edge_index1)
    for l in range(L):
        sa = _spmv_call(ha, src3a, dst3a)
        sb = _spmv_call(hb, src3b, dst3b)
        ha = _layer_call(sa, aedega, We[l], be[l], W1[l], b1[l],
                         W2[l], b2[l], relu_out=(l < L - 1))
        hb = _layer_call(sb, aedegb, We[l], be[l], W1[l], b1[l],
                         W2[l], b2[l], relu_out=(l < L - 1))
    pos = _finish(ha, batch0, Wc1, bc1, Wc2, bc2)
    neg = _finish(hb, batch1, Wc1, bc1, Wc2, bc2)
    logits = jnp.concatenate([pos, neg], axis=0)
    labels = jnp.concatenate([jnp.ones((G,), jnp.float32),
                              jnp.zeros((G,), jnp.float32)], axis=0)
    return logits, labels
